# baseline (speedup 1.0000x reference)
import functools

import jax
import jax.numpy as jnp
from jax import lax
from jax.experimental import pallas as pl
from jax.experimental.pallas import tpu as pltpu
from jax.experimental.pallas import tpu_sc as plsc

_N = 10000
_E = 160000
_H = 32
_NW = 32
_EPW = _E // _NW
_CH = 125
_NCH = _EPW // _CH
_CPP = 8
_PH = _NCH // _CPP
_EB = 640
_NB = 1000


def _ew_body(ef_ref, w1_ref, b1_ref, w2_ref, b2_ref, out_ref):
    hid = jnp.maximum(
        jnp.dot(ef_ref[...], w1_ref[...], preferred_element_type=jnp.float32)
        + b1_ref[...], 0.0)
    out_ref[...] = (
        jnp.dot(hid, w2_ref[...], preferred_element_type=jnp.float32)
        + b2_ref[...])


def _compute_ew(edge_feats, We1, be1, We2, be2):
    return pl.pallas_call(
        _ew_body,
        grid=(_E // _EB,),
        in_specs=[
            pl.BlockSpec((_EB, 16), lambda i: (i, 0)),
            pl.BlockSpec((16, 128), lambda i: (0, 0)),
            pl.BlockSpec((1, 128), lambda i: (0, 0)),
            pl.BlockSpec((128, 1024), lambda i: (0, 0)),
            pl.BlockSpec((1, 1024), lambda i: (0, 0)),
        ],
        out_specs=pl.BlockSpec((_EB, 1024), lambda i: (i, 0)),
        out_shape=jax.ShapeDtypeStruct((_E, 1024), jnp.float32),
    )(edge_feats, We1, be1.reshape(1, -1), We2, be2.reshape(1, -1))


def _msg_body(ew_ref, xs_ref, out_ref):
    xs = xs_ref[...]
    ew = ew_ref[...]
    acc = xs[:, 0:1] * ew[:, 0:_H]
    for i in range(1, _H):
        acc = acc + xs[:, i:i + 1] * ew[:, i * _H:(i + 1) * _H]
    out_ref[...] = acc


def _compute_msg(ew, xs):
    return pl.pallas_call(
        _msg_body,
        grid=(_E // _EB,),
        in_specs=[
            pl.BlockSpec((_EB, 1024), lambda i: (i, 0)),
            pl.BlockSpec((_EB, _H), lambda i: (i, 0)),
        ],
        out_specs=pl.BlockSpec((_EB, _H), lambda i: (i, 0)),
        out_shape=jax.ShapeDtypeStruct((_E, _H), jnp.float32),
    )(ew, xs)


def _gru_body(agg_ref, h_ref, wih_ref, whh_ref, bih_ref, bhh_ref, nnb_ref,
              out_ref):
    x = jnp.maximum(agg_ref[0] + agg_ref[1] + nnb_ref[...], 0.0)
    h = h_ref[...]
    gi = jnp.dot(x, wih_ref[...], preferred_element_type=jnp.float32) + bih_ref[...]
    gh = jnp.dot(h, whh_ref[...], preferred_element_type=jnp.float32) + bhh_ref[...]
    r = jax.nn.sigmoid(gi[:, :_H] + gh[:, :_H])
    z = jax.nn.sigmoid(gi[:, _H:2 * _H] + gh[:, _H:2 * _H])
    n = jnp.tanh(gi[:, 2 * _H:] + r * gh[:, 2 * _H:])
    out_ref[...] = (1.0 - z) * n + z * h


def _gru(agg_parts, h, WihT, WhhT, b_ih, b_hh, nn_bias):
    return pl.pallas_call(
        _gru_body,
        grid=(_N // _NB,),
        in_specs=[
            pl.BlockSpec((2, _NB, _H), lambda i: (0, i, 0)),
            pl.BlockSpec((_NB, _H), lambda i: (i, 0)),
            pl.BlockSpec((_H, 3 * _H), lambda i: (0, 0)),
            pl.BlockSpec((_H, 3 * _H), lambda i: (0, 0)),
            pl.BlockSpec((1, 3 * _H), lambda i: (0, 0)),
            pl.BlockSpec((1, 3 * _H), lambda i: (0, 0)),
            pl.BlockSpec((1, _H), lambda i: (0, 0)),
        ],
        out_specs=pl.BlockSpec((_NB, _H), lambda i: (i, 0)),
        out_shape=jax.ShapeDtypeStruct((_N, _H), jnp.float32),
    )(agg_parts, h, WihT, WhhT, b_ih.reshape(1, -1), b_hh.reshape(1, -1),
      nn_bias.reshape(1, -1))


def _mlp_body(x_ref, w1_ref, b1_ref, w2_ref, b2_ref, out_ref):
    t = jnp.maximum(
        jnp.dot(x_ref[...], w1_ref[...], preferred_element_type=jnp.float32)
        + b1_ref[...], 0.0)
    out_ref[...] = (
        jnp.dot(t, w2_ref[...], preferred_element_type=jnp.float32)
        + b2_ref[...])


def _mlp(x, W1, b1, W2, b2):
    d_in, d_mid = W1.shape
    d_out = W2.shape[1]
    return pl.pallas_call(
        _mlp_body,
        grid=(_N // _NB,),
        in_specs=[
            pl.BlockSpec((_NB, d_in), lambda i: (i, 0)),
            pl.BlockSpec((d_in, d_mid), lambda i: (0, 0)),
            pl.BlockSpec((1, d_mid), lambda i: (0, 0)),
            pl.BlockSpec((d_mid, d_out), lambda i: (0, 0)),
            pl.BlockSpec((1, d_out), lambda i: (0, 0)),
        ],
        out_specs=pl.BlockSpec((_NB, d_out), lambda i: (i, 0)),
        out_shape=jax.ShapeDtypeStruct((_N, d_out), jnp.float32),
    )(x, W1, b1.reshape(1, -1), W2, b2.reshape(1, -1))


def _sc_gather(x, src3):
    mesh = plsc.VectorSubcoreMesh(core_axis_name="c", subcore_axis_name="s")

    @functools.partial(
        pl.kernel,
        out_type=jax.ShapeDtypeStruct((_E, _H), jnp.float32),
        mesh=mesh,
        scratch_types=[
            pltpu.VMEM((_NCH, _CH), jnp.int32),
            pltpu.VMEM((_CPP * _CH, _H), jnp.float32),
            pltpu.SemaphoreType.DMA,
        ],
        compiler_params=pltpu.CompilerParams(use_tc_tiling_on_sc=False),
    )
    def body(x_hbm, src_hbm, out_hbm, idx_v, rows_v, sem):
        cid = lax.axis_index("c")
        sid = lax.axis_index("s")
        wid = cid * 16 + sid
        pltpu.sync_copy(src_hbm.at[wid], idx_v)
        for p in range(_PH):
            cps = []
            for b in range(_CPP):
                cps.append(pltpu.async_copy(
                    x_hbm.at[idx_v.at[p * _CPP + b]],
                    rows_v.at[pl.ds(b * _CH, _CH)], sem))
            for cp in cps:
                cp.wait()
            pltpu.sync_copy(
                rows_v,
                out_hbm.at[pl.ds(wid * _EPW + p * _CPP * _CH, _CPP * _CH)])

    return body(x, src3)


def _sc_scatter(msg, dst3, zeros):
    mesh = plsc.VectorSubcoreMesh(core_axis_name="c", subcore_axis_name="s")
    nrows = 1000

    @functools.partial(
        pl.kernel,
        out_type=jax.ShapeDtypeStruct((2, _N, _H), jnp.float32),
        mesh=mesh,
        scratch_types=[
            pltpu.VMEM((_NCH, _CH), jnp.int32),
            pltpu.VMEM((_CPP * _CH, _H), jnp.float32),
            pltpu.VMEM_SHARED((_N, _H), jnp.float32),
            pltpu.SemaphoreType.DMA,
        ],
        compiler_params=pltpu.CompilerParams(use_tc_tiling_on_sc=False),
    )
    def body(msg_hbm, dst_hbm, z_hbm, out_hbm, idx_v, buf_v, acc_sh, sem):
        cid = lax.axis_index("c")
        sid = lax.axis_index("s")
        wid = cid * 16 + sid

        @pl.when(sid < 10)
        def _init():
            pltpu.sync_copy(z_hbm.at[pl.ds(sid * nrows, nrows)],
                            acc_sh.at[pl.ds(sid * nrows, nrows)])

        pltpu.sync_copy(dst_hbm.at[wid], idx_v)
        plsc.subcore_barrier()
        for p in range(_PH):
            pltpu.sync_copy(
                msg_hbm.at[pl.ds(wid * _EPW + p * _CPP * _CH, _CPP * _CH)],
                buf_v)
            cps = []
            for b in range(_CPP):
                cps.append(pltpu.async_copy(
                    buf_v.at[pl.ds(b * _CH, _CH)],
                    acc_sh.at[idx_v.at[p * _CPP + b]], sem, add=True))
            for cp in cps:
                cp.wait()
        plsc.subcore_barrier()

        @pl.when(sid < 10)
        def _writeout():
            pltpu.sync_copy(acc_sh.at[pl.ds(sid * nrows, nrows)],
                            out_hbm.at[cid, pl.ds(sid * nrows, nrows)])

    return body(msg, dst3, zeros)


def kernel(node_feats, edge_feats, edge_index, W_proj1, b_proj1, W_proj2,
           b_proj2, We1, be1, We2, be2, nn_bias, W_ih, W_hh, b_ih, b_hh,
           Wd1, bd1, Wd2, bd2):
    src3 = edge_index[0].reshape(_NW, _NCH, _CH)
    dst3 = edge_index[1].reshape(_NW, _NCH, _CH)
    zeros = jnp.zeros((_N, _H), jnp.float32)
    WihT = W_ih.T
    WhhT = W_hh.T

    ew = _compute_ew(edge_feats, We1, be1, We2, be2)
    x = _mlp(node_feats, W_proj1, b_proj1, W_proj2, b_proj2)
    h = x
    for _ in range(3):
        xs = _sc_gather(x, src3)
        msg = _compute_msg(ew, xs)
        agg_parts = _sc_scatter(msg, dst3, zeros)
        h = _gru(agg_parts, h, WihT, WhhT, b_ih, b_hh, nn_bias)
        x = h
    return _mlp(h, Wd1, bd1, Wd2, bd2)


# speedup vs baseline: 2.5328x; 2.5328x over previous
import functools

import jax
import jax.numpy as jnp
from jax import lax
from jax.experimental import pallas as pl
from jax.experimental.pallas import tpu as pltpu
from jax.experimental.pallas import tpu_sc as plsc

_N = 10000
_E = 160000
_H = 32
_NW = 32
_EPW = _E // _NW
_CH = 125
_NCH = _EPW // _CH
_CPP = 8
_PH = _NCH // _CPP
_EB = 640
_NB = 1000


def _ew_body(ef_ref, w1_ref, b1_ref, w2_ref, b2_ref, out_ref):
    hid = jnp.maximum(
        jnp.dot(ef_ref[...], w1_ref[...], preferred_element_type=jnp.float32)
        + b1_ref[...], 0.0)
    out_ref[...] = (
        jnp.dot(hid, w2_ref[...], preferred_element_type=jnp.float32)
        + b2_ref[...])


def _compute_ew(edge_feats, We1, be1, We2, be2):
    return pl.pallas_call(
        _ew_body,
        grid=(_E // _EB,),
        in_specs=[
            pl.BlockSpec((_EB, 16), lambda i: (i, 0)),
            pl.BlockSpec((16, 128), lambda i: (0, 0)),
            pl.BlockSpec((1, 128), lambda i: (0, 0)),
            pl.BlockSpec((128, 1024), lambda i: (0, 0)),
            pl.BlockSpec((1, 1024), lambda i: (0, 0)),
        ],
        out_specs=pl.BlockSpec((_EB, 1024), lambda i: (i, 0)),
        out_shape=jax.ShapeDtypeStruct((_E, 1024), jnp.float32),
    )(edge_feats, We1, be1.reshape(1, -1), We2, be2.reshape(1, -1))


def _msg_body(ew_ref, xs_ref, r_ref, s_ref, out_ref):
    xs_rep = jnp.dot(xs_ref[...], r_ref[...], preferred_element_type=jnp.float32)
    prod = xs_rep * ew_ref[...]
    acc = prod[:, 0:128]
    for j in range(1, 8):
        acc = acc + prod[:, 128 * j:128 * (j + 1)]
    out_ref[...] = jnp.dot(acc, s_ref[...], preferred_element_type=jnp.float32)


def _compute_msg(ew, xs, R, S):
    return pl.pallas_call(
        _msg_body,
        grid=(_E // _EB,),
        in_specs=[
            pl.BlockSpec((_EB, 1024), lambda i: (i, 0)),
            pl.BlockSpec((_EB, _H), lambda i: (i, 0)),
            pl.BlockSpec((_H, 1024), lambda i: (0, 0)),
            pl.BlockSpec((128, _H), lambda i: (0, 0)),
        ],
        out_specs=pl.BlockSpec((_EB, _H), lambda i: (i, 0)),
        out_shape=jax.ShapeDtypeStruct((_E, _H), jnp.float32),
    )(ew, xs, R, S)


def _gru_body(agg_ref, h_ref, wih_ref, whh_ref, bih_ref, bhh_ref, nnb_ref,
              out_ref):
    x = jnp.maximum(agg_ref[0] + agg_ref[1] + nnb_ref[...], 0.0)
    h = h_ref[...]
    gi = jnp.dot(x, wih_ref[...], preferred_element_type=jnp.float32) + bih_ref[...]
    gh = jnp.dot(h, whh_ref[...], preferred_element_type=jnp.float32) + bhh_ref[...]
    r = jax.nn.sigmoid(gi[:, :_H] + gh[:, :_H])
    z = jax.nn.sigmoid(gi[:, _H:2 * _H] + gh[:, _H:2 * _H])
    n = jnp.tanh(gi[:, 2 * _H:] + r * gh[:, 2 * _H:])
    out_ref[...] = (1.0 - z) * n + z * h


def _gru(agg_parts, h, WihT, WhhT, b_ih, b_hh, nn_bias):
    return pl.pallas_call(
        _gru_body,
        grid=(_N // _NB,),
        in_specs=[
            pl.BlockSpec((2, _NB, _H), lambda i: (0, i, 0)),
            pl.BlockSpec((_NB, _H), lambda i: (i, 0)),
            pl.BlockSpec((_H, 3 * _H), lambda i: (0, 0)),
            pl.BlockSpec((_H, 3 * _H), lambda i: (0, 0)),
            pl.BlockSpec((1, 3 * _H), lambda i: (0, 0)),
            pl.BlockSpec((1, 3 * _H), lambda i: (0, 0)),
            pl.BlockSpec((1, _H), lambda i: (0, 0)),
        ],
        out_specs=pl.BlockSpec((_NB, _H), lambda i: (i, 0)),
        out_shape=jax.ShapeDtypeStruct((_N, _H), jnp.float32),
    )(agg_parts, h, WihT, WhhT, b_ih.reshape(1, -1), b_hh.reshape(1, -1),
      nn_bias.reshape(1, -1))


def _mlp_body(x_ref, w1_ref, b1_ref, w2_ref, b2_ref, out_ref):
    t = jnp.maximum(
        jnp.dot(x_ref[...], w1_ref[...], preferred_element_type=jnp.float32)
        + b1_ref[...], 0.0)
    out_ref[...] = (
        jnp.dot(t, w2_ref[...], preferred_element_type=jnp.float32)
        + b2_ref[...])


def _mlp(x, W1, b1, W2, b2):
    d_in, d_mid = W1.shape
    d_out = W2.shape[1]
    return pl.pallas_call(
        _mlp_body,
        grid=(_N // _NB,),
        in_specs=[
            pl.BlockSpec((_NB, d_in), lambda i: (i, 0)),
            pl.BlockSpec((d_in, d_mid), lambda i: (0, 0)),
            pl.BlockSpec((1, d_mid), lambda i: (0, 0)),
            pl.BlockSpec((d_mid, d_out), lambda i: (0, 0)),
            pl.BlockSpec((1, d_out), lambda i: (0, 0)),
        ],
        out_specs=pl.BlockSpec((_NB, d_out), lambda i: (i, 0)),
        out_shape=jax.ShapeDtypeStruct((_N, d_out), jnp.float32),
    )(x, W1, b1.reshape(1, -1), W2, b2.reshape(1, -1))


def _sc_gather(x, src3):
    mesh = plsc.VectorSubcoreMesh(core_axis_name="c", subcore_axis_name="s")

    @functools.partial(
        pl.kernel,
        out_type=jax.ShapeDtypeStruct((_E, _H), jnp.float32),
        mesh=mesh,
        scratch_types=[
            pltpu.VMEM((_NCH, _CH), jnp.int32),
            pltpu.VMEM((_CPP * _CH, _H), jnp.float32),
            pltpu.SemaphoreType.DMA,
        ],
        compiler_params=pltpu.CompilerParams(use_tc_tiling_on_sc=False),
    )
    def body(x_hbm, src_hbm, out_hbm, idx_v, rows_v, sem):
        cid = lax.axis_index("c")
        sid = lax.axis_index("s")
        wid = cid * 16 + sid
        pltpu.sync_copy(src_hbm.at[wid], idx_v)
        for p in range(_PH):
            cps = []
            for b in range(_CPP):
                cps.append(pltpu.async_copy(
                    x_hbm.at[idx_v.at[p * _CPP + b]],
                    rows_v.at[pl.ds(b * _CH, _CH)], sem))
            for cp in cps:
                cp.wait()
            pltpu.sync_copy(
                rows_v,
                out_hbm.at[pl.ds(wid * _EPW + p * _CPP * _CH, _CPP * _CH)])

    return body(x, src3)


def _sc_scatter(msg, dst3, zeros):
    mesh = plsc.VectorSubcoreMesh(core_axis_name="c", subcore_axis_name="s")
    nrows = 1000

    @functools.partial(
        pl.kernel,
        out_type=jax.ShapeDtypeStruct((2, _N, _H), jnp.float32),
        mesh=mesh,
        scratch_types=[
            pltpu.VMEM((_NCH, _CH), jnp.int32),
            pltpu.VMEM((_CPP * _CH, _H), jnp.float32),
            pltpu.VMEM_SHARED((_N, _H), jnp.float32),
            pltpu.SemaphoreType.DMA,
        ],
        compiler_params=pltpu.CompilerParams(use_tc_tiling_on_sc=False),
    )
    def body(msg_hbm, dst_hbm, z_hbm, out_hbm, idx_v, buf_v, acc_sh, sem):
        cid = lax.axis_index("c")
        sid = lax.axis_index("s")
        wid = cid * 16 + sid

        @pl.when(sid < 10)
        def _init():
            pltpu.sync_copy(z_hbm.at[pl.ds(sid * nrows, nrows)],
                            acc_sh.at[pl.ds(sid * nrows, nrows)])

        pltpu.sync_copy(dst_hbm.at[wid], idx_v)
        plsc.subcore_barrier()
        for p in range(_PH):
            pltpu.sync_copy(
                msg_hbm.at[pl.ds(wid * _EPW + p * _CPP * _CH, _CPP * _CH)],
                buf_v)
            cps = []
            for b in range(_CPP):
                cps.append(pltpu.async_copy(
                    buf_v.at[pl.ds(b * _CH, _CH)],
                    acc_sh.at[idx_v.at[p * _CPP + b]], sem, add=True))
            for cp in cps:
                cp.wait()
        plsc.subcore_barrier()

        @pl.when(sid < 10)
        def _writeout():
            pltpu.sync_copy(acc_sh.at[pl.ds(sid * nrows, nrows)],
                            out_hbm.at[cid, pl.ds(sid * nrows, nrows)])

    return body(msg, dst3, zeros)


def kernel(node_feats, edge_feats, edge_index, W_proj1, b_proj1, W_proj2,
           b_proj2, We1, be1, We2, be2, nn_bias, W_ih, W_hh, b_ih, b_hh,
           Wd1, bd1, Wd2, bd2):
    src3 = edge_index[0].reshape(_NW, _NCH, _CH)
    dst3 = edge_index[1].reshape(_NW, _NCH, _CH)
    zeros = jnp.zeros((_N, _H), jnp.float32)
    WihT = W_ih.T
    WhhT = W_hh.T
    R = (jnp.arange(1024)[None, :] // _H
         == jnp.arange(_H)[:, None]).astype(jnp.float32)
    S = (jnp.arange(128)[:, None] % _H
         == jnp.arange(_H)[None, :]).astype(jnp.float32)

    ew = _compute_ew(edge_feats, We1, be1, We2, be2)
    x = _mlp(node_feats, W_proj1, b_proj1, W_proj2, b_proj2)
    h = x
    for _ in range(3):
        xs = _sc_gather(x, src3)
        msg = _compute_msg(ew, xs, R, S)
        agg_parts = _sc_scatter(msg, dst3, zeros)
        h = _gru(agg_parts, h, WihT, WhhT, b_ih, b_hh, nn_bias)
        x = h
    return _mlp(h, Wd1, bd1, Wd2, bd2)


# speedup vs baseline: 2.8328x; 1.1185x over previous
import functools

import jax
import jax.numpy as jnp
from jax import lax
from jax.experimental import pallas as pl
from jax.experimental.pallas import tpu as pltpu
from jax.experimental.pallas import tpu_sc as plsc

_N = 10000
_E = 160000
_H = 32
_NW = 32
_EPW = _E // _NW
_CH = 125
_NCH = _EPW // _CH
_CPP = 8
_PH = _NCH // _CPP
_EB = 640
_NB = 1000


def _ew_body(ef_ref, w1_ref, b1_ref, w2_ref, b2_ref, out_ref):
    hid = jnp.maximum(
        jnp.dot(ef_ref[...], w1_ref[...], preferred_element_type=jnp.float32)
        + b1_ref[...], 0.0)
    out_ref[...] = (
        jnp.dot(hid, w2_ref[...], preferred_element_type=jnp.float32)
        + b2_ref[...]).astype(jnp.bfloat16)


def _compute_ew(edge_feats, We1, be1, We2, be2):
    return pl.pallas_call(
        _ew_body,
        grid=(_E // _EB,),
        in_specs=[
            pl.BlockSpec((_EB, 16), lambda i: (i, 0)),
            pl.BlockSpec((16, 128), lambda i: (0, 0)),
            pl.BlockSpec((1, 128), lambda i: (0, 0)),
            pl.BlockSpec((128, 1024), lambda i: (0, 0)),
            pl.BlockSpec((1, 1024), lambda i: (0, 0)),
        ],
        out_specs=pl.BlockSpec((_EB, 1024), lambda i: (i, 0)),
        out_shape=jax.ShapeDtypeStruct((_E, 1024), jnp.bfloat16),
    )(edge_feats, We1, be1.reshape(1, -1), We2, be2.reshape(1, -1))


def _msg_body(ew_ref, xs_ref, r_ref, s_ref, out_ref):
    xs_rep = jnp.dot(xs_ref[...], r_ref[...], preferred_element_type=jnp.float32)
    prod = xs_rep * ew_ref[...].astype(jnp.float32)
    acc = prod[:, 0:128]
    for j in range(1, 8):
        acc = acc + prod[:, 128 * j:128 * (j + 1)]
    out_ref[...] = jnp.dot(acc, s_ref[...], preferred_element_type=jnp.float32)


def _compute_msg(ew, xs, R, S):
    return pl.pallas_call(
        _msg_body,
        grid=(_E // _EB,),
        in_specs=[
            pl.BlockSpec((_EB, 1024), lambda i: (i, 0)),
            pl.BlockSpec((_EB, _H), lambda i: (i, 0)),
            pl.BlockSpec((_H, 1024), lambda i: (0, 0)),
            pl.BlockSpec((128, _H), lambda i: (0, 0)),
        ],
        out_specs=pl.BlockSpec((_EB, _H), lambda i: (i, 0)),
        out_shape=jax.ShapeDtypeStruct((_E, _H), jnp.float32),
    )(ew, xs, R, S)


def _gru_body(agg_ref, h_ref, wih_ref, whh_ref, bih_ref, bhh_ref, nnb_ref,
              out_ref):
    x = jnp.maximum(agg_ref[0] + agg_ref[1] + nnb_ref[...], 0.0)
    h = h_ref[...]
    gi = jnp.dot(x, wih_ref[...], preferred_element_type=jnp.float32) + bih_ref[...]
    gh = jnp.dot(h, whh_ref[...], preferred_element_type=jnp.float32) + bhh_ref[...]
    r = jax.nn.sigmoid(gi[:, :_H] + gh[:, :_H])
    z = jax.nn.sigmoid(gi[:, _H:2 * _H] + gh[:, _H:2 * _H])
    n = jnp.tanh(gi[:, 2 * _H:] + r * gh[:, 2 * _H:])
    out_ref[...] = (1.0 - z) * n + z * h


def _gru(agg_parts, h, WihT, WhhT, b_ih, b_hh, nn_bias):
    return pl.pallas_call(
        _gru_body,
        grid=(_N // _NB,),
        in_specs=[
            pl.BlockSpec((2, _NB, _H), lambda i: (0, i, 0)),
            pl.BlockSpec((_NB, _H), lambda i: (i, 0)),
            pl.BlockSpec((_H, 3 * _H), lambda i: (0, 0)),
            pl.BlockSpec((_H, 3 * _H), lambda i: (0, 0)),
            pl.BlockSpec((1, 3 * _H), lambda i: (0, 0)),
            pl.BlockSpec((1, 3 * _H), lambda i: (0, 0)),
            pl.BlockSpec((1, _H), lambda i: (0, 0)),
        ],
        out_specs=pl.BlockSpec((_NB, _H), lambda i: (i, 0)),
        out_shape=jax.ShapeDtypeStruct((_N, _H), jnp.float32),
    )(agg_parts, h, WihT, WhhT, b_ih.reshape(1, -1), b_hh.reshape(1, -1),
      nn_bias.reshape(1, -1))


def _mlp_body(x_ref, w1_ref, b1_ref, w2_ref, b2_ref, out_ref):
    t = jnp.maximum(
        jnp.dot(x_ref[...], w1_ref[...], preferred_element_type=jnp.float32)
        + b1_ref[...], 0.0)
    out_ref[...] = (
        jnp.dot(t, w2_ref[...], preferred_element_type=jnp.float32)
        + b2_ref[...])


def _mlp(x, W1, b1, W2, b2):
    d_in, d_mid = W1.shape
    d_out = W2.shape[1]
    return pl.pallas_call(
        _mlp_body,
        grid=(_N // _NB,),
        in_specs=[
            pl.BlockSpec((_NB, d_in), lambda i: (i, 0)),
            pl.BlockSpec((d_in, d_mid), lambda i: (0, 0)),
            pl.BlockSpec((1, d_mid), lambda i: (0, 0)),
            pl.BlockSpec((d_mid, d_out), lambda i: (0, 0)),
            pl.BlockSpec((1, d_out), lambda i: (0, 0)),
        ],
        out_specs=pl.BlockSpec((_NB, d_out), lambda i: (i, 0)),
        out_shape=jax.ShapeDtypeStruct((_N, d_out), jnp.float32),
    )(x, W1, b1.reshape(1, -1), W2, b2.reshape(1, -1))


def _sc_gather(x, src3):
    mesh = plsc.VectorSubcoreMesh(core_axis_name="c", subcore_axis_name="s")

    @functools.partial(
        pl.kernel,
        out_type=jax.ShapeDtypeStruct((_E, _H), jnp.float32),
        mesh=mesh,
        scratch_types=[
            pltpu.VMEM((_NCH, _CH), jnp.int32),
            pltpu.VMEM((_CPP * _CH, _H), jnp.float32),
            pltpu.SemaphoreType.DMA,
        ],
        compiler_params=pltpu.CompilerParams(use_tc_tiling_on_sc=False),
    )
    def body(x_hbm, src_hbm, out_hbm, idx_v, rows_v, sem):
        cid = lax.axis_index("c")
        sid = lax.axis_index("s")
        wid = cid * 16 + sid
        pltpu.sync_copy(src_hbm.at[wid], idx_v)
        for p in range(_PH):
            cps = []
            for b in range(_CPP):
                cps.append(pltpu.async_copy(
                    x_hbm.at[idx_v.at[p * _CPP + b]],
                    rows_v.at[pl.ds(b * _CH, _CH)], sem))
            for cp in cps:
                cp.wait()
            pltpu.sync_copy(
                rows_v,
                out_hbm.at[pl.ds(wid * _EPW + p * _CPP * _CH, _CPP * _CH)])

    return body(x, src3)


def _sc_scatter(msg, dst3, zeros):
    mesh = plsc.VectorSubcoreMesh(core_axis_name="c", subcore_axis_name="s")
    nrows = 1000

    @functools.partial(
        pl.kernel,
        out_type=jax.ShapeDtypeStruct((2, _N, _H), jnp.float32),
        mesh=mesh,
        scratch_types=[
            pltpu.VMEM((_NCH, _CH), jnp.int32),
            pltpu.VMEM((_CPP * _CH, _H), jnp.float32),
            pltpu.VMEM_SHARED((_N, _H), jnp.float32),
            pltpu.SemaphoreType.DMA,
        ],
        compiler_params=pltpu.CompilerParams(use_tc_tiling_on_sc=False),
    )
    def body(msg_hbm, dst_hbm, z_hbm, out_hbm, idx_v, buf_v, acc_sh, sem):
        cid = lax.axis_index("c")
        sid = lax.axis_index("s")
        wid = cid * 16 + sid

        @pl.when(sid < 10)
        def _init():
            pltpu.sync_copy(z_hbm.at[pl.ds(sid * nrows, nrows)],
                            acc_sh.at[pl.ds(sid * nrows, nrows)])

        pltpu.sync_copy(dst_hbm.at[wid], idx_v)
        plsc.subcore_barrier()
        for p in range(_PH):
            pltpu.sync_copy(
                msg_hbm.at[pl.ds(wid * _EPW + p * _CPP * _CH, _CPP * _CH)],
                buf_v)
            cps = []
            for b in range(_CPP):
                cps.append(pltpu.async_copy(
                    buf_v.at[pl.ds(b * _CH, _CH)],
                    acc_sh.at[idx_v.at[p * _CPP + b]], sem, add=True))
            for cp in cps:
                cp.wait()
        plsc.subcore_barrier()

        @pl.when(sid < 10)
        def _writeout():
            pltpu.sync_copy(acc_sh.at[pl.ds(sid * nrows, nrows)],
                            out_hbm.at[cid, pl.ds(sid * nrows, nrows)])

    return body(msg, dst3, zeros)


def kernel(node_feats, edge_feats, edge_index, W_proj1, b_proj1, W_proj2,
           b_proj2, We1, be1, We2, be2, nn_bias, W_ih, W_hh, b_ih, b_hh,
           Wd1, bd1, Wd2, bd2):
    src3 = edge_index[0].reshape(_NW, _NCH, _CH)
    dst3 = edge_index[1].reshape(_NW, _NCH, _CH)
    zeros = jnp.zeros((_N, _H), jnp.float32)
    WihT = W_ih.T
    WhhT = W_hh.T
    R = (jnp.arange(1024)[None, :] // _H
         == jnp.arange(_H)[:, None]).astype(jnp.float32)
    S = (jnp.arange(128)[:, None] % _H
         == jnp.arange(_H)[None, :]).astype(jnp.float32)

    ew = _compute_ew(edge_feats, We1, be1, We2, be2)
    x = _mlp(node_feats, W_proj1, b_proj1, W_proj2, b_proj2)
    h = x
    for _ in range(3):
        xs = _sc_gather(x, src3)
        msg = _compute_msg(ew, xs, R, S)
        agg_parts = _sc_scatter(msg, dst3, zeros)
        h = _gru(agg_parts, h, WihT, WhhT, b_ih, b_hh, nn_bias)
        x = h
    return _mlp(h, Wd1, bd1, Wd2, bd2)
